# baseline (device time: 12095 ns/iter reference)
import jax
import jax.numpy as jnp
from jax import lax
from jax.experimental import pallas as pl
from jax.experimental.pallas import tpu as pltpu

N_DEV = 4
EPS = 1e-5
N_CHUNK = 2


def kernel(x, t_emb, W_scale, W_shift):
    b, s, c_local = x.shape
    c_global = c_local * N_DEV
    cs = s // N_CHUNK

    def body(x_hbm, t_hbm, ws_hbm, wsh_hbm, dummy_hbm, out_hbm,
             x_vmem, t_vmem, w_vmem, out_vmem, comm_ref,
             in_sems, out_sems, send_sems, recv_sems):
        del dummy_hbm
        my_pos = lax.axis_index("i")

        barrier_sem = pltpu.get_barrier_semaphore()
        for k in range(1, N_DEV):
            pl.semaphore_signal(
                barrier_sem, inc=1,
                device_id=((my_pos + k) % N_DEV,),
                device_id_type=pl.DeviceIdType.MESH,
            )

        x_copies = []
        for j in range(N_CHUNK):
            cp = pltpu.make_async_copy(
                x_hbm.at[:, pl.ds(j * cs, cs), :],
                x_vmem.at[:, pl.ds(j * cs, cs), :],
                in_sems.at[j],
            )
            cp.start()
            x_copies.append(cp)
        t_copy = pltpu.make_async_copy(t_hbm, t_vmem, in_sems.at[N_CHUNK])
        ws_copy = pltpu.make_async_copy(ws_hbm, w_vmem.at[0], in_sems.at[N_CHUNK + 1])
        wsh_copy = pltpu.make_async_copy(wsh_hbm, w_vmem.at[1], in_sems.at[N_CHUNK + 2])
        t_copy.start()
        ws_copy.start()
        wsh_copy.start()

        chunk_rdmas = []
        for j in range(N_CHUNK):
            x_copies[j].wait()
            xj = x_vmem[:, pl.ds(j * cs, cs), :]
            s1 = jnp.sum(xj, axis=-1)
            s2 = jnp.sum(xj * xj, axis=-1)
            comm_ref[j, 0] = jnp.stack([s1, s2])
            if j == 0:
                pl.semaphore_wait(barrier_sem, N_DEV - 1)
            sends = []
            for k in range(1, N_DEV):
                rdma = pltpu.make_async_remote_copy(
                    src_ref=comm_ref.at[j, 0],
                    dst_ref=comm_ref.at[j, k],
                    send_sem=send_sems.at[j, k - 1],
                    recv_sem=recv_sems.at[j, k - 1],
                    device_id=((my_pos + k) % N_DEV,),
                    device_id_type=pl.DeviceIdType.MESH,
                )
                rdma.start()
                sends.append(rdma)
            chunk_rdmas.append(sends)

        t_copy.wait()
        ws_copy.wait()
        wsh_copy.wait()
        tv = t_vmem[:, :]
        scale = jnp.dot(tv, w_vmem[0], preferred_element_type=jnp.float32)
        shift = jnp.dot(tv, w_vmem[1], preferred_element_type=jnp.float32)
        scale1 = (1.0 + scale)[:, None, :]
        shift1 = shift[:, None, :]

        out_copies = []
        for j in range(N_CHUNK):
            for rdma in chunk_rdmas[j]:
                rdma.wait_recv()
            stats = (comm_ref[j, 0] + comm_ref[j, 1]) + (
                comm_ref[j, 2] + comm_ref[j, 3])
            mean = stats[0] / c_global
            var = stats[1] / c_global - mean * mean
            inv = lax.rsqrt(var + EPS)
            xj = x_vmem[:, pl.ds(j * cs, cs), :]
            h = (xj - mean[:, :, None]) * inv[:, :, None]
            out = h * scale1 + shift1
            out_vmem[:, pl.ds(j * cs, cs), :] = out.astype(jnp.bfloat16)
            cp = pltpu.make_async_copy(
                out_vmem.at[:, pl.ds(j * cs, cs), :],
                out_hbm.at[:, pl.ds(j * cs, cs), :],
                out_sems.at[j],
            )
            cp.start()
            out_copies.append(cp)

        for cp in out_copies:
            cp.wait()
        for sends in chunk_rdmas:
            for rdma in sends:
                rdma.wait_send()

    t_dim = t_emb.shape[1]

    x = pltpu.with_memory_space_constraint(x, pltpu.MemorySpace.HBM)
    t_emb = pltpu.with_memory_space_constraint(t_emb, pltpu.MemorySpace.HBM)
    W_scale = pltpu.with_memory_space_constraint(W_scale, pltpu.MemorySpace.HBM)
    W_shift = pltpu.with_memory_space_constraint(W_shift, pltpu.MemorySpace.HBM)

    dummy = pltpu.with_memory_space_constraint(
        jnp.zeros((b, s, c_local), jnp.bfloat16), pltpu.MemorySpace.HBM)

    return pl.pallas_call(
        body,
        out_shape=jax.ShapeDtypeStruct((b, s, c_local), jnp.bfloat16),
        in_specs=[pl.BlockSpec(memory_space=pltpu.MemorySpace.HBM)] * 5,
        out_specs=pl.BlockSpec(memory_space=pltpu.MemorySpace.HBM),
        input_output_aliases={4: 0},
        scratch_shapes=[
            pltpu.VMEM((b, s, c_local), jnp.float32),
            pltpu.VMEM((b, t_dim), jnp.float32),
            pltpu.VMEM((2, t_dim, c_local), jnp.float32),
            pltpu.VMEM((b, s, c_local), jnp.bfloat16),
            pltpu.VMEM((N_CHUNK, N_DEV, 2, b, cs), jnp.float32),
            pltpu.SemaphoreType.DMA((N_CHUNK + 3,)),
            pltpu.SemaphoreType.DMA((N_CHUNK,)),
            pltpu.SemaphoreType.DMA((N_CHUNK, N_DEV - 1)),
            pltpu.SemaphoreType.DMA((N_CHUNK, N_DEV - 1)),
        ],
        compiler_params=pltpu.CompilerParams(collective_id=0),
    )(x, t_emb, W_scale, W_shift, dummy)


# device time: 8885 ns/iter; 1.3613x vs baseline; 1.3613x over previous
import jax
import jax.numpy as jnp
from jax import lax
from jax.experimental import pallas as pl
from jax.experimental.pallas import tpu as pltpu

N_DEV = 4
EPS = 1e-5
N_CHUNK = 4


def kernel(x, t_emb, W_scale, W_shift):
    b, s, c_local = x.shape
    c_global = c_local * N_DEV
    cs = s // N_CHUNK

    def body(x_hbm, t_hbm, ws_hbm, wsh_hbm, out_hbm,
             x_vmem, t_vmem, w_vmem, out_vmem, comm_ref,
             in_sems, out_sems, send_sems, recv_sems):
        my_pos = lax.axis_index("i")

        barrier_sem = pltpu.get_barrier_semaphore()
        for k in range(1, N_DEV):
            pl.semaphore_signal(
                barrier_sem, inc=1,
                device_id=((my_pos + k) % N_DEV,),
                device_id_type=pl.DeviceIdType.MESH,
            )

        x_copies = []
        for j in range(N_CHUNK):
            cp = pltpu.make_async_copy(
                x_hbm.at[:, pl.ds(j * cs, cs), :],
                x_vmem.at[:, pl.ds(j * cs, cs), :],
                in_sems.at[j],
            )
            cp.start()
            x_copies.append(cp)
        t_copy = pltpu.make_async_copy(t_hbm, t_vmem, in_sems.at[N_CHUNK])
        ws_copy = pltpu.make_async_copy(ws_hbm, w_vmem.at[0], in_sems.at[N_CHUNK + 1])
        wsh_copy = pltpu.make_async_copy(wsh_hbm, w_vmem.at[1], in_sems.at[N_CHUNK + 2])
        t_copy.start()
        ws_copy.start()
        wsh_copy.start()

        chunk_rdmas = []
        for j in range(N_CHUNK):
            x_copies[j].wait()
            xj = x_vmem[:, pl.ds(j * cs, cs), :]
            s1 = jnp.sum(xj, axis=-1)
            s2 = jnp.sum(xj * xj, axis=-1)
            comm_ref[j, 0] = jnp.stack([s1, s2])
            if j == 0:
                pl.semaphore_wait(barrier_sem, N_DEV - 1)
            sends = []
            for k in range(1, N_DEV):
                rdma = pltpu.make_async_remote_copy(
                    src_ref=comm_ref.at[j, 0],
                    dst_ref=comm_ref.at[j, k],
                    send_sem=send_sems.at[j, k - 1],
                    recv_sem=recv_sems.at[j, k - 1],
                    device_id=((my_pos + k) % N_DEV,),
                    device_id_type=pl.DeviceIdType.MESH,
                )
                rdma.start()
                sends.append(rdma)
            chunk_rdmas.append(sends)

        t_copy.wait()
        ws_copy.wait()
        wsh_copy.wait()
        tv = t_vmem[:, :]
        scale = jnp.dot(tv, w_vmem[0], preferred_element_type=jnp.float32)
        shift = jnp.dot(tv, w_vmem[1], preferred_element_type=jnp.float32)
        scale1 = (1.0 + scale)[:, None, :]
        shift1 = shift[:, None, :]

        out_copies = []
        for j in range(N_CHUNK):
            for rdma in chunk_rdmas[j]:
                rdma.wait_recv()
            stats = (comm_ref[j, 0] + comm_ref[j, 1]) + (
                comm_ref[j, 2] + comm_ref[j, 3])
            mean = stats[0] / c_global
            var = stats[1] / c_global - mean * mean
            inv = lax.rsqrt(var + EPS)
            xj = x_vmem[:, pl.ds(j * cs, cs), :]
            h = (xj - mean[:, :, None]) * inv[:, :, None]
            out = h * scale1 + shift1
            out_vmem[:, pl.ds(j * cs, cs), :] = out.astype(jnp.bfloat16)
            cp = pltpu.make_async_copy(
                out_vmem.at[:, pl.ds(j * cs, cs), :],
                out_hbm.at[:, pl.ds(j * cs, cs), :],
                out_sems.at[j],
            )
            cp.start()
            out_copies.append(cp)

        for cp in out_copies:
            cp.wait()
        for sends in chunk_rdmas:
            for rdma in sends:
                rdma.wait_send()

    t_dim = t_emb.shape[1]

    x = pltpu.with_memory_space_constraint(x, pltpu.MemorySpace.HBM)
    t_emb = pltpu.with_memory_space_constraint(t_emb, pltpu.MemorySpace.HBM)
    W_scale = pltpu.with_memory_space_constraint(W_scale, pltpu.MemorySpace.HBM)
    W_shift = pltpu.with_memory_space_constraint(W_shift, pltpu.MemorySpace.HBM)

    return pl.pallas_call(
        body,
        out_shape=jax.ShapeDtypeStruct((b, s, c_local), jnp.bfloat16),
        in_specs=[pl.BlockSpec(memory_space=pltpu.MemorySpace.HBM)] * 4,
        out_specs=pl.BlockSpec(memory_space=pltpu.MemorySpace.HBM),
        scratch_shapes=[
            pltpu.VMEM((b, s, c_local), jnp.float32),
            pltpu.VMEM((b, t_dim), jnp.float32),
            pltpu.VMEM((2, t_dim, c_local), jnp.float32),
            pltpu.VMEM((b, s, c_local), jnp.bfloat16),
            pltpu.VMEM((N_CHUNK, N_DEV, 2, b, cs), jnp.float32),
            pltpu.SemaphoreType.DMA((N_CHUNK + 3,)),
            pltpu.SemaphoreType.DMA((N_CHUNK,)),
            pltpu.SemaphoreType.DMA((N_CHUNK, N_DEV - 1)),
            pltpu.SemaphoreType.DMA((N_CHUNK, N_DEV - 1)),
        ],
        compiler_params=pltpu.CompilerParams(collective_id=0),
    )(x, t_emb, W_scale, W_shift)


# device time: 8808 ns/iter; 1.3732x vs baseline; 1.0087x over previous
import jax
import jax.numpy as jnp
from jax import lax
from jax.experimental import pallas as pl
from jax.experimental.pallas import tpu as pltpu

N_DEV = 4
EPS = 1e-5
N_CHUNK = 4


def kernel(x, t_emb, W_scale, W_shift):
    b, s, c_local = x.shape
    c_global = c_local * N_DEV
    cs = s // N_CHUNK

    def body(x_hbm, t_hbm, ws_hbm, wsh_hbm, out_ref,
             x_vmem, t_vmem, w_vmem, comm_ref,
             in_sems, send_sems, recv_sems):
        my_pos = lax.axis_index("i")

        barrier_sem = pltpu.get_barrier_semaphore()
        for k in range(1, N_DEV):
            pl.semaphore_signal(
                barrier_sem, inc=1,
                device_id=((my_pos + k) % N_DEV,),
                device_id_type=pl.DeviceIdType.MESH,
            )

        x_copies = []
        for j in range(N_CHUNK):
            x_copies.append(pltpu.make_async_copy(
                x_hbm.at[:, pl.ds(j * cs, cs), :],
                x_vmem.at[:, pl.ds(j * cs, cs), :],
                in_sems.at[j],
            ))
        t_copy = pltpu.make_async_copy(t_hbm, t_vmem, in_sems.at[N_CHUNK])
        ws_copy = pltpu.make_async_copy(ws_hbm, w_vmem.at[0], in_sems.at[N_CHUNK + 1])
        wsh_copy = pltpu.make_async_copy(wsh_hbm, w_vmem.at[1], in_sems.at[N_CHUNK + 2])
        x_copies[0].start()
        t_copy.start()
        ws_copy.start()
        wsh_copy.start()
        for j in range(1, N_CHUNK):
            x_copies[j].start()

        chunk_rdmas = []
        for j in range(N_CHUNK):
            x_copies[j].wait()
            xj = x_vmem[:, pl.ds(j * cs, cs), :]
            s1 = jnp.sum(xj, axis=-1)
            s2 = jnp.sum(xj * xj, axis=-1)
            comm_ref[j, 0] = jnp.stack([s1, s2])
            if j == 0:
                pl.semaphore_wait(barrier_sem, N_DEV - 1)
            sends = []
            for k in range(1, N_DEV):
                rdma = pltpu.make_async_remote_copy(
                    src_ref=comm_ref.at[j, 0],
                    dst_ref=comm_ref.at[j, k],
                    send_sem=send_sems.at[j, k - 1],
                    recv_sem=recv_sems.at[j, k - 1],
                    device_id=((my_pos + k) % N_DEV,),
                    device_id_type=pl.DeviceIdType.MESH,
                )
                rdma.start()
                sends.append(rdma)
            chunk_rdmas.append(sends)

        t_copy.wait()
        ws_copy.wait()
        wsh_copy.wait()
        tv = t_vmem[:, :]
        scale = jnp.dot(tv, w_vmem[0], preferred_element_type=jnp.float32)
        shift = jnp.dot(tv, w_vmem[1], preferred_element_type=jnp.float32)
        scale1 = (1.0 + scale)[:, None, :]
        shift1 = shift[:, None, :]

        for j in range(N_CHUNK):
            for rdma in chunk_rdmas[j]:
                rdma.wait_recv()
            stats = (comm_ref[j, 0] + comm_ref[j, 1]) + (
                comm_ref[j, 2] + comm_ref[j, 3])
            mean = stats[0] / c_global
            var = stats[1] / c_global - mean * mean
            inv = lax.rsqrt(var + EPS)
            xj = x_vmem[:, pl.ds(j * cs, cs), :]
            h = (xj - mean[:, :, None]) * inv[:, :, None]
            out = h * scale1 + shift1
            out_ref[:, pl.ds(j * cs, cs), :] = out.astype(jnp.bfloat16)

        for sends in chunk_rdmas:
            for rdma in sends:
                rdma.wait_send()

    t_dim = t_emb.shape[1]

    x = pltpu.with_memory_space_constraint(x, pltpu.MemorySpace.HBM)
    t_emb = pltpu.with_memory_space_constraint(t_emb, pltpu.MemorySpace.HBM)
    W_scale = pltpu.with_memory_space_constraint(W_scale, pltpu.MemorySpace.HBM)
    W_shift = pltpu.with_memory_space_constraint(W_shift, pltpu.MemorySpace.HBM)

    return pl.pallas_call(
        body,
        out_shape=jax.ShapeDtypeStruct((b, s, c_local), jnp.bfloat16),
        in_specs=[pl.BlockSpec(memory_space=pltpu.MemorySpace.HBM)] * 4,
        out_specs=pl.BlockSpec(memory_space=pltpu.VMEM),
        scratch_shapes=[
            pltpu.VMEM((b, s, c_local), jnp.float32),
            pltpu.VMEM((b, t_dim), jnp.float32),
            pltpu.VMEM((2, t_dim, c_local), jnp.float32),
            pltpu.VMEM((N_CHUNK, N_DEV, 2, b, cs), jnp.float32),
            pltpu.SemaphoreType.DMA((N_CHUNK + 3,)),
            pltpu.SemaphoreType.DMA((N_CHUNK, N_DEV - 1)),
            pltpu.SemaphoreType.DMA((N_CHUNK, N_DEV - 1)),
        ],
        compiler_params=pltpu.CompilerParams(collective_id=0),
    )(x, t_emb, W_scale, W_shift)
